# baseline (speedup 1.0000x reference)
import functools

import jax
import jax.numpy as jnp
from jax import lax
from jax.experimental import pallas as pl
from jax.experimental.pallas import tpu as pltpu
from jax.experimental.pallas import tpu_sc as plsc

NUM_ROWS = 1_000_000
D = 64
B_TOTAL = 16384 * 26
NW = 32
B_PER_W = B_TOTAL // NW
CHUNK = 128
N_CHUNKS = B_PER_W // CHUNK


def _emb_kernel(idx_hbm, table_hbm, out_hbm, idx_v, rows_v, idx_sem, sem):
    wid = lax.axis_index("s") * 2 + lax.axis_index("c")
    base = wid * B_PER_W
    pltpu.async_copy(idx_hbm.at[wid], idx_v, idx_sem).wait()

    def body(j, carry):
        pltpu.async_copy(table_hbm.at[idx_v.at[j]], rows_v, sem).wait()
        pltpu.sync_copy(rows_v, out_hbm.at[pl.ds(base + j * CHUNK, CHUNK)])
        return carry

    lax.fori_loop(0, N_CHUNKS, body, 0)


@jax.jit
def kernel(Z, weight):
    idx = Z.reshape(NW, N_CHUNKS, CHUNK).astype(jnp.int32)
    mesh = plsc.VectorSubcoreMesh(core_axis_name="c", subcore_axis_name="s")
    out = pl.kernel(
        _emb_kernel,
        out_type=jax.ShapeDtypeStruct((B_TOTAL, D), jnp.float32),
        mesh=mesh,
        scratch_types=[
            pltpu.VMEM((N_CHUNKS, CHUNK), jnp.int32),
            pltpu.VMEM((CHUNK, D), jnp.float32),
            pltpu.SemaphoreType.DMA,
            pltpu.SemaphoreType.DMA,
        ],
        compiler_params=pltpu.CompilerParams(use_tc_tiling_on_sc=False),
    )(idx, weight)
    return out.reshape(16384, 26, D)


# speedup vs baseline: 1.0793x; 1.0793x over previous
import functools

import jax
import jax.numpy as jnp
from jax import lax
from jax.experimental import pallas as pl
from jax.experimental.pallas import tpu as pltpu
from jax.experimental.pallas import tpu_sc as plsc

NUM_ROWS = 1_000_000
D = 64
B_TOTAL = 16384 * 26
NW = 32
B_PER_W = B_TOTAL // NW
CHUNK = 128
N_CHUNKS = B_PER_W // CHUNK


NBUF = 8
N_GROUPS = N_CHUNKS // NBUF


def _emb_kernel(idx_hbm, table_hbm, out_hbm, idx_v, rows, gsems, ssems,
                idx_sem):
    wid = lax.axis_index("s") * 2 + lax.axis_index("c")
    base = wid * B_PER_W
    pltpu.async_copy(idx_hbm.at[wid], idx_v, idx_sem).wait()

    def gather_refs(j, b):
        return (table_hbm.at[idx_v.at[j]], rows.at[b], gsems.at[b])

    def store_refs(j, b):
        return (rows.at[b], out_hbm.at[pl.ds(base + j * CHUNK, CHUNK)],
                ssems.at[b])

    for b in range(NBUF):
        pltpu.async_copy(*gather_refs(b, b))

    def group(g, carry):
        for b in range(NBUF):
            j = g * NBUF + b
            pltpu.make_async_copy(*gather_refs(j, b)).wait()
            pltpu.async_copy(*store_refs(j, b))

            @pl.when(g < N_GROUPS - 1)
            def _():
                pltpu.make_async_copy(*store_refs(j, b)).wait()
                pltpu.async_copy(*gather_refs(j + NBUF, b))
        return carry

    lax.fori_loop(0, N_GROUPS, group, 0)

    for b in range(NBUF):
        j = (N_GROUPS - 1) * NBUF + b
        pltpu.make_async_copy(*store_refs(j, b)).wait()


@jax.jit
def kernel(Z, weight):
    idx = Z.reshape(NW, N_CHUNKS, CHUNK).astype(jnp.int32)
    mesh = plsc.VectorSubcoreMesh(core_axis_name="c", subcore_axis_name="s")
    out = pl.kernel(
        _emb_kernel,
        out_type=jax.ShapeDtypeStruct((B_TOTAL, D), jnp.float32),
        mesh=mesh,
        scratch_types=[
            pltpu.VMEM((N_CHUNKS, CHUNK), jnp.int32),
            pltpu.VMEM((NBUF, CHUNK, D), jnp.float32),
            pltpu.SemaphoreType.DMA((NBUF,)),
            pltpu.SemaphoreType.DMA((NBUF,)),
            pltpu.SemaphoreType.DMA,
        ],
        compiler_params=pltpu.CompilerParams(use_tc_tiling_on_sc=False),
    )(idx, weight)
    return out.reshape(16384, 26, D)
